# baseline (speedup 1.0000x reference)
import functools

import jax
import jax.numpy as jnp
from jax import lax
from jax.experimental import pallas as pl
from jax.experimental.pallas import tpu as pltpu
from jax.experimental.pallas import tpu_sc as plsc

N = 10000
NP = 10240
H = 256
HH = 128
E = 160000
EP = 163840
TILES = 16
CH = 80
CW = 128
GR = 8
CNT_W = 8
MP = 3072
BLK = 400


def _tc_encode(x, W_encT, a11, dec, masked2d):
    def body(x_ref, w_ref, a_ref, dec_ref, m_ref, ha_ref, hb_ref):
        i = pl.program_id(0)
        a = a_ref[0, 0]
        xb = x_ref[...]
        act = jnp.where(xb >= 0, xb, a * xb)
        h = jnp.dot(act, w_ref[...], preferred_element_type=jnp.float32)
        rid = lax.broadcasted_iota(jnp.int32, (BLK, 1), 0) + i * BLK
        acc = jnp.zeros((BLK, 128), dtype=jnp.bool_)
        for k in range(MP // 128):
            mk = m_ref[k:k + 1, :]
            acc = acc | (rid == mk)
        flag = jnp.any(acc, axis=1, keepdims=True)
        h = jnp.where(flag, dec_ref[...], h)
        ha_ref[...] = h[:, :HH]
        hb_ref[...] = h[:, HH:]

    return pl.pallas_call(
        body,
        grid=(N // BLK,),
        in_specs=[
            pl.BlockSpec((BLK, H), lambda i: (i, 0)),
            pl.BlockSpec((H, H), lambda i: (0, 0)),
            pl.BlockSpec((1, 1), lambda i: (0, 0)),
            pl.BlockSpec((1, H), lambda i: (0, 0)),
            pl.BlockSpec((MP // 128, 128), lambda i: (0, 0)),
        ],
        out_specs=[
            pl.BlockSpec((BLK, HH), lambda i: (i, 0)),
            pl.BlockSpec((BLK, HH), lambda i: (i, 0)),
        ],
        out_shape=[
            jax.ShapeDtypeStruct((N, HH), jnp.float32),
            jax.ShapeDtypeStruct((N, HH), jnp.float32),
        ],
    )(x, W_encT, a11, dec, masked2d)


def _sc_aggregate(ha, hb, src3, dst3, t3, d3):
    mesh = plsc.VectorSubcoreMesh(core_axis_name="c", subcore_axis_name="s")

    @functools.partial(
        pl.kernel,
        mesh=mesh,
        out_type=[
            jax.ShapeDtypeStruct((N, HH), jnp.float32),
            jax.ShapeDtypeStruct((N, HH), jnp.float32),
            jax.ShapeDtypeStruct((N * CNT_W,), jnp.float32),
            jax.ShapeDtypeStruct((N * CNT_W,), jnp.float32),
        ],
        scratch_types=[
            pltpu.VMEM((GR, CW), jnp.int32),
            pltpu.VMEM((GR, CW), jnp.int32),
            pltpu.VMEM((GR, CW), jnp.int32),
            pltpu.VMEM((GR, CW), jnp.int32),
            pltpu.VMEM((GR, CW), jnp.int32),
            pltpu.VMEM((GR, CW), jnp.int32),
            pltpu.SemaphoreType.DMA,
            pltpu.VMEM((CW, HH), jnp.float32),
            pltpu.VMEM((CW, HH), jnp.float32),
            pltpu.VMEM((2, CW), jnp.int32),
            pltpu.VMEM((CW,), jnp.float32),
            pltpu.VMEM((1024,), jnp.float32),
            pltpu.SemaphoreType.DMA,
            pltpu.SemaphoreType.DMA,
            pltpu.SemaphoreType.DMA,
            pltpu.SemaphoreType.DMA,
            pltpu.SemaphoreType.DMA,
            pltpu.SemaphoreType.DMA,
            pltpu.SemaphoreType.DMA,
            pltpu.VMEM_SHARED((NP, HH), jnp.float32),
            pltpu.VMEM_SHARED((NP * CNT_W,), jnp.float32),
        ],
    )
    def k(ha_hbm, hb_hbm, src_hbm, dst_hbm, t_hbm, d_hbm,
          aggr_a_hbm, aggr_b_hbm, cnt0_hbm, cnt1_hbm,
          srcA_v, dstA_v, aA_v, srcB_v, dstB_v, aB_v, semiB,
          rows0_v, rows1_v, cidx_v, ones_v, zero1_v,
          sem0, sem1, sem2, sem3, sem4, sem5, semi, aggr_s, cnt_s):
        c = lax.axis_index("c")
        s = lax.axis_index("s")

        z16 = jnp.zeros((16,), jnp.float32)
        def fill2(i, _):
            for g in range(HH // 16):
                rows0_v[i, slice(g * 16, g * 16 + 16)] = z16
            return 0
        lax.fori_loop(0, CW, fill2, 0)
        def fill1(i, _):
            zero1_v[pl.ds(i * 16, 16)] = z16
            return 0
        lax.fori_loop(0, 64, fill1, 0)
        for g in range(CW // 16):
            ones_v[pl.ds(g * 16, 16)] = jnp.ones((16,), jnp.float32)

        rows_per_tile = NP // TILES
        def preload(h_hbm):
            @pl.when(s < TILES - 1)
            def _():
                rsl = pl.ds(s * rows_per_tile, rows_per_tile)
                pltpu.sync_copy(h_hbm.at[rsl], aggr_s.at[rsl])
            @pl.when(s == TILES - 1)
            def _():
                last = N - (TILES - 1) * rows_per_tile
                base = (TILES - 1) * rows_per_tile
                pltpu.sync_copy(h_hbm.at[pl.ds(base, last)],
                                aggr_s.at[pl.ds(base, last)])
                pltpu.sync_copy(rows0_v, aggr_s.at[pl.ds(N, CW)])
                pltpu.sync_copy(rows0_v.at[pl.ds(0, NP - N - CW)],
                                aggr_s.at[pl.ds(N + CW, NP - N - CW)])
        @pl.when(c == 0)
        def _():
            preload(ha_hbm)
        @pl.when(c == 1)
        def _():
            preload(hb_hbm)
        cnt_per_tile = NP * CNT_W // TILES
        for r in range(cnt_per_tile // 1024):
            pltpu.sync_copy(
                zero1_v, cnt_s.at[pl.ds(s * cnt_per_tile + r * 1024, 1024)])

        plsc.subcore_barrier()

        def edge_phase(h_hbm, attr_hbm):
            bufs = (rows0_v, rows1_v)
            gsem = (sem0, sem1)
            ssem = (sem2, sem3)
            csem = (sem4, sem5)
            idxA = (srcA_v, dstA_v, aA_v)
            idxB = (srcB_v, dstB_v, aB_v)

            def stage(gsl, bufset, sem):
                pltpu.async_copy(src_hbm.at[s, gsl], bufset[0], sem)
                pltpu.async_copy(dst_hbm.at[s, gsl], bufset[1], sem)
                pltpu.async_copy(attr_hbm.at[s, gsl], bufset[2], sem)

            def wait_stage(bufset, sem):
                z = pl.ds(0, GR)
                pltpu.make_async_copy(src_hbm.at[s, z], bufset[0], sem).wait()
                pltpu.make_async_copy(dst_hbm.at[s, z], bufset[1], sem).wait()
                pltpu.make_async_copy(attr_hbm.at[s, z], bufset[2], sem).wait()

            def proc(bufset):
                src_v, dst_v, a_v = bufset
                cg = [None, None]
                cs = [None, None]
                cc = [None, None]
                cg[0] = pltpu.async_copy(
                    h_hbm.at[src_v.at[0]], bufs[0], gsem[0])
                for j in range(GR):
                    b = j & 1
                    cg[b].wait()
                    cs[b] = pltpu.async_copy(
                        bufs[b], aggr_s.at[dst_v.at[j]], ssem[b], add=True)
                    if j + 1 < GR:
                        if cs[1 - b] is not None:
                            cs[1 - b].wait()
                        cg[1 - b] = pltpu.async_copy(
                            h_hbm.at[src_v.at[j + 1]], bufs[1 - b],
                            gsem[1 - b])
                    if cc[b] is not None:
                        cc[b].wait()
                    for g in range(CW // 16):
                        sl = slice(g * 16, g * 16 + 16)
                        cidx_v[b, sl] = dst_v[j, sl] * CNT_W + a_v[j, sl]
                    cc[b] = pltpu.async_copy(
                        ones_v, cnt_s.at[cidx_v.at[b]], csem[b], add=True)
                for cp in (cs[0], cs[1], cc[0], cc[1]):
                    if cp is not None:
                        cp.wait()

            NB = CH // GR // 2
            stage(pl.ds(0, GR), idxA, semi)
            def body(gi, _):
                g0 = 2 * gi
                wait_stage(idxA, semi)
                stage(pl.ds((g0 + 1) * GR, GR), idxB, semiB)
                proc(idxA)
                wait_stage(idxB, semiB)
                @pl.when(gi < NB - 1)
                def _():
                    stage(pl.ds((g0 + 2) * GR, GR), idxA, semi)
                proc(idxB)
                return 0
            lax.fori_loop(0, NB, body, 0)

        @pl.when(c == 0)
        def _():
            edge_phase(ha_hbm, t_hbm)
        @pl.when(c == 1)
        def _():
            edge_phase(hb_hbm, d_hbm)

        plsc.subcore_barrier()

        def dump(aggr_hbm, cnt_hbm):
            full = NP // TILES
            cfull = full * CNT_W
            @pl.when(s < TILES - 1)
            def _():
                rsl = pl.ds(s * full, full)
                csl = pl.ds(s * cfull, cfull)
                pltpu.sync_copy(aggr_s.at[rsl], aggr_hbm.at[rsl])
                pltpu.sync_copy(cnt_s.at[csl], cnt_hbm.at[csl])
            @pl.when(s == TILES - 1)
            def _():
                last = N - (TILES - 1) * full
                rsl = pl.ds((TILES - 1) * full, last)
                csl = pl.ds((TILES - 1) * cfull, last * CNT_W)
                pltpu.sync_copy(aggr_s.at[rsl], aggr_hbm.at[rsl])
                pltpu.sync_copy(cnt_s.at[csl], cnt_hbm.at[csl])

        @pl.when(c == 0)
        def _():
            dump(aggr_a_hbm, cnt0_hbm)
        @pl.when(c == 1)
        def _():
            dump(aggr_b_hbm, cnt1_hbm)

    return k(ha, hb, src3, dst3, t3, d3)


def _tc_mlp(aggr_a, aggr_b, cnt0, cnt1, T0, T1, selfemb,
            W1T, b1r, W2T, b2r, WoutT, bor):
    def body(aa_ref, ab_ref, c0_ref, c1_ref, t0_ref, t1_ref,
             se_ref, w1_ref, b1_ref, w2_ref, b2_ref, wo_ref, bo_ref, o_ref,
             t0w_ref, t1w_ref, sb_ref, wc_ref, bc_ref):
        i = pl.program_id(0)

        @pl.when(i == 0)
        def _():
            w1 = w1_ref[...]
            t0w_ref[...] = jnp.dot(t0_ref[...], w1,
                                   preferred_element_type=jnp.float32)
            t1w_ref[...] = jnp.dot(t1_ref[...], w1,
                                   preferred_element_type=jnp.float32)
            sb_ref[...] = jnp.dot(se_ref[...], w1,
                                  preferred_element_type=jnp.float32) \
                + b1_ref[...]
            wo = wo_ref[...]
            wc_ref[...] = jnp.dot(w2_ref[...], wo,
                                  preferred_element_type=jnp.float32)
            bc_ref[...] = jnp.dot(b2_ref[...], wo,
                                  preferred_element_type=jnp.float32) \
                + bo_ref[...]

        g = jnp.dot(aa_ref[...], w1_ref[0:HH, :],
                    preferred_element_type=jnp.float32)
        g = g + jnp.dot(ab_ref[...], w1_ref[HH:H, :],
                        preferred_element_type=jnp.float32)
        g = g + jnp.dot(c0_ref[...], t0w_ref[...],
                        preferred_element_type=jnp.float32)
        g = g + jnp.dot(c1_ref[...], t1w_ref[...],
                        preferred_element_type=jnp.float32)
        g = jnp.maximum(g + sb_ref[...], 0.0)
        o_ref[...] = jnp.dot(g, wc_ref[...],
                             preferred_element_type=jnp.float32) + bc_ref[...]

    return pl.pallas_call(
        body,
        grid=(N // BLK,),
        in_specs=[
            pl.BlockSpec((BLK, HH), lambda i: (i, 0)),
            pl.BlockSpec((BLK, HH), lambda i: (i, 0)),
            pl.BlockSpec((BLK, CNT_W), lambda i: (i, 0)),
            pl.BlockSpec((BLK, CNT_W), lambda i: (i, 0)),
            pl.BlockSpec((CNT_W, H), lambda i: (0, 0)),
            pl.BlockSpec((CNT_W, H), lambda i: (0, 0)),
            pl.BlockSpec((1, H), lambda i: (0, 0)),
            pl.BlockSpec((H, 2 * H), lambda i: (0, 0)),
            pl.BlockSpec((1, 2 * H), lambda i: (0, 0)),
            pl.BlockSpec((2 * H, H), lambda i: (0, 0)),
            pl.BlockSpec((1, H), lambda i: (0, 0)),
            pl.BlockSpec((H, H), lambda i: (0, 0)),
            pl.BlockSpec((1, H), lambda i: (0, 0)),
        ],
        out_specs=pl.BlockSpec((BLK, H), lambda i: (i, 0)),
        out_shape=jax.ShapeDtypeStruct((N, H), jnp.float32),
        scratch_shapes=[
            pltpu.VMEM((CNT_W, 2 * H), jnp.float32),
            pltpu.VMEM((CNT_W, 2 * H), jnp.float32),
            pltpu.VMEM((1, 2 * H), jnp.float32),
            pltpu.VMEM((2 * H, H), jnp.float32),
            pltpu.VMEM((1, H), jnp.float32),
        ],
    )(aggr_a, aggr_b, cnt0, cnt1, T0, T1, selfemb,
      W1T, b1r, W2T, b2r, WoutT, bor)


def kernel(x, edge_index, edge_attr, batch, masked_node_indices, W_enc,
           prelu_a, dec_token, emb1, emb2, W1, b1, W2, b2, W_out, b_out):
    W_encT = W_enc.T
    a11 = jnp.reshape(prelu_a, (1, 1))
    masked2d = jnp.pad(masked_node_indices,
                       (0, MP - masked_node_indices.shape[0]),
                       constant_values=-1).reshape(MP // 128, 128)

    src = edge_index[0]
    dst = edge_index[1]
    pad_n = EP - E
    pad_iota = jnp.arange(pad_n, dtype=jnp.int32)
    src_p = jnp.concatenate([src, pad_iota % N])
    dst_p = jnp.concatenate([dst, N + (pad_iota % (NP - N))])
    t_p = jnp.concatenate([edge_attr[:, 0], jnp.zeros(pad_n, jnp.int32)])
    d_p = jnp.concatenate([edge_attr[:, 1], jnp.zeros(pad_n, jnp.int32)])
    src3 = src_p.reshape(TILES, CH, CW)
    dst3 = dst_p.reshape(TILES, CH, CW)
    t3 = t_p.reshape(TILES, CH, CW)
    d3 = d_p.reshape(TILES, CH, CW)

    T0 = jnp.zeros((CNT_W, H), jnp.float32).at[0:6].set(emb1)
    T1 = jnp.zeros((CNT_W, H), jnp.float32).at[0:3].set(emb2)
    selfemb = (emb1[4] + emb2[0]).reshape(1, H)

    W1T = W1.T
    b1r = b1.reshape(1, 2 * H)
    W2T = W2.T
    b2r = b2.reshape(1, H)
    WoutT = W_out.T
    bor = b_out.reshape(1, H)

    ha, hb = _tc_encode(x, W_encT, a11, dec_token, masked2d)

    aggr_a, aggr_b, cnt0_flat, cnt1_flat = _sc_aggregate(
        ha, hb, src3, dst3, t3, d3)
    cnt0 = cnt0_flat.reshape(N, CNT_W)
    cnt1 = cnt1_flat.reshape(N, CNT_W)

    return _tc_mlp(aggr_a, aggr_b, cnt0, cnt1, T0, T1, selfemb,
                   W1T, b1r, W2T, b2r, WoutT, bor)


# speedup vs baseline: 1.0039x; 1.0039x over previous
import functools

import jax
import jax.numpy as jnp
from jax import lax
from jax.experimental import pallas as pl
from jax.experimental.pallas import tpu as pltpu
from jax.experimental.pallas import tpu_sc as plsc

N = 10000
NP = 10240
H = 256
HH = 128
E = 160000
EP = 163840
TILES = 16
CH = 80
CW = 128
GR = 16
CNT_W = 8
MP = 3072
BLK = 400


def _tc_encode(x, W_encT, a11, dec, masked2d):
    def body(x_ref, w_ref, a_ref, dec_ref, m_ref, ha_ref, hb_ref):
        i = pl.program_id(0)
        a = a_ref[0, 0]
        xb = x_ref[...]
        act = jnp.where(xb >= 0, xb, a * xb)
        h = jnp.dot(act, w_ref[...], preferred_element_type=jnp.float32)
        rid = lax.broadcasted_iota(jnp.int32, (BLK, 1), 0) + i * BLK
        acc = jnp.zeros((BLK, 128), dtype=jnp.bool_)
        for k in range(MP // 128):
            mk = m_ref[k:k + 1, :]
            acc = acc | (rid == mk)
        flag = jnp.any(acc, axis=1, keepdims=True)
        h = jnp.where(flag, dec_ref[...], h)
        ha_ref[...] = h[:, :HH]
        hb_ref[...] = h[:, HH:]

    return pl.pallas_call(
        body,
        grid=(N // BLK,),
        in_specs=[
            pl.BlockSpec((BLK, H), lambda i: (i, 0)),
            pl.BlockSpec((H, H), lambda i: (0, 0)),
            pl.BlockSpec((1, 1), lambda i: (0, 0)),
            pl.BlockSpec((1, H), lambda i: (0, 0)),
            pl.BlockSpec((MP // 128, 128), lambda i: (0, 0)),
        ],
        out_specs=[
            pl.BlockSpec((BLK, HH), lambda i: (i, 0)),
            pl.BlockSpec((BLK, HH), lambda i: (i, 0)),
        ],
        out_shape=[
            jax.ShapeDtypeStruct((N, HH), jnp.float32),
            jax.ShapeDtypeStruct((N, HH), jnp.float32),
        ],
    )(x, W_encT, a11, dec, masked2d)


def _sc_aggregate(ha, hb, src3, dst3, t3, d3):
    mesh = plsc.VectorSubcoreMesh(core_axis_name="c", subcore_axis_name="s")

    @functools.partial(
        pl.kernel,
        mesh=mesh,
        out_type=[
            jax.ShapeDtypeStruct((N, HH), jnp.float32),
            jax.ShapeDtypeStruct((N, HH), jnp.float32),
            jax.ShapeDtypeStruct((N * CNT_W,), jnp.float32),
            jax.ShapeDtypeStruct((N * CNT_W,), jnp.float32),
        ],
        scratch_types=[
            pltpu.VMEM((GR, CW), jnp.int32),
            pltpu.VMEM((GR, CW), jnp.int32),
            pltpu.VMEM((GR, CW), jnp.int32),
            pltpu.VMEM((CW, HH), jnp.float32),
            pltpu.VMEM((CW, HH), jnp.float32),
            pltpu.VMEM((2, CW), jnp.int32),
            pltpu.VMEM((CW,), jnp.float32),
            pltpu.VMEM((1024,), jnp.float32),
            pltpu.SemaphoreType.DMA,
            pltpu.SemaphoreType.DMA,
            pltpu.SemaphoreType.DMA,
            pltpu.SemaphoreType.DMA,
            pltpu.SemaphoreType.DMA,
            pltpu.SemaphoreType.DMA,
            pltpu.SemaphoreType.DMA,
            pltpu.VMEM_SHARED((NP, HH), jnp.float32),
            pltpu.VMEM_SHARED((NP * CNT_W,), jnp.float32),
        ],
    )
    def k(ha_hbm, hb_hbm, src_hbm, dst_hbm, t_hbm, d_hbm,
          aggr_a_hbm, aggr_b_hbm, cnt0_hbm, cnt1_hbm,
          src_v, dst_v, a_v, rows0_v, rows1_v, cidx_v, ones_v, zero1_v,
          sem0, sem1, sem2, sem3, sem4, sem5, semi, aggr_s, cnt_s):
        c = lax.axis_index("c")
        s = lax.axis_index("s")

        z16 = jnp.zeros((16,), jnp.float32)
        def fill2(i, _):
            for g in range(HH // 16):
                rows0_v[i, slice(g * 16, g * 16 + 16)] = z16
            return 0
        lax.fori_loop(0, CW, fill2, 0)
        def fill1(i, _):
            zero1_v[pl.ds(i * 16, 16)] = z16
            return 0
        lax.fori_loop(0, 64, fill1, 0)
        for g in range(CW // 16):
            ones_v[pl.ds(g * 16, 16)] = jnp.ones((16,), jnp.float32)

        rows_per_tile = NP // TILES
        def preload(h_hbm):
            @pl.when(s < TILES - 1)
            def _():
                rsl = pl.ds(s * rows_per_tile, rows_per_tile)
                pltpu.sync_copy(h_hbm.at[rsl], aggr_s.at[rsl])
            @pl.when(s == TILES - 1)
            def _():
                last = N - (TILES - 1) * rows_per_tile
                base = (TILES - 1) * rows_per_tile
                pltpu.sync_copy(h_hbm.at[pl.ds(base, last)],
                                aggr_s.at[pl.ds(base, last)])
                pltpu.sync_copy(rows0_v, aggr_s.at[pl.ds(N, CW)])
                pltpu.sync_copy(rows0_v.at[pl.ds(0, NP - N - CW)],
                                aggr_s.at[pl.ds(N + CW, NP - N - CW)])
        @pl.when(c == 0)
        def _():
            preload(ha_hbm)
        @pl.when(c == 1)
        def _():
            preload(hb_hbm)
        cnt_per_tile = NP * CNT_W // TILES
        for r in range(cnt_per_tile // 1024):
            pltpu.sync_copy(
                zero1_v, cnt_s.at[pl.ds(s * cnt_per_tile + r * 1024, 1024)])

        plsc.subcore_barrier()

        def edge_phase(h_hbm, attr_hbm):
            bufs = (rows0_v, rows1_v)
            gsem = (sem0, sem1)
            ssem = (sem2, sem3)
            csem = (sem4, sem5)
            def group(gr, _):
                gsl = pl.ds(gr * GR, GR)
                i1 = pltpu.async_copy(src_hbm.at[s, gsl], src_v, semi)
                i2 = pltpu.async_copy(dst_hbm.at[s, gsl], dst_v, semi)
                i3 = pltpu.async_copy(attr_hbm.at[s, gsl], a_v, semi)
                i1.wait(); i2.wait(); i3.wait()
                cg = [None, None]
                cs = [None, None]
                cc = [None, None]
                cg[0] = pltpu.async_copy(
                    h_hbm.at[src_v.at[0]], bufs[0], gsem[0])
                for j in range(GR):
                    b = j & 1
                    cg[b].wait()
                    cs[b] = pltpu.async_copy(
                        bufs[b], aggr_s.at[dst_v.at[j]], ssem[b], add=True)
                    if j + 1 < GR:
                        if cs[1 - b] is not None:
                            cs[1 - b].wait()
                        cg[1 - b] = pltpu.async_copy(
                            h_hbm.at[src_v.at[j + 1]], bufs[1 - b],
                            gsem[1 - b])
                    if cc[b] is not None:
                        cc[b].wait()
                    for g in range(CW // 16):
                        sl = slice(g * 16, g * 16 + 16)
                        cidx_v[b, sl] = dst_v[j, sl] * CNT_W + a_v[j, sl]
                    cc[b] = pltpu.async_copy(
                        ones_v, cnt_s.at[cidx_v.at[b]], csem[b], add=True)
                for cp in (cs[0], cs[1], cc[0], cc[1]):
                    if cp is not None:
                        cp.wait()
                return 0
            lax.fori_loop(0, CH // GR, group, 0)

        @pl.when(c == 0)
        def _():
            edge_phase(ha_hbm, t_hbm)
        @pl.when(c == 1)
        def _():
            edge_phase(hb_hbm, d_hbm)

        plsc.subcore_barrier()

        def dump(aggr_hbm, cnt_hbm):
            full = NP // TILES
            cfull = full * CNT_W
            @pl.when(s < TILES - 1)
            def _():
                rsl = pl.ds(s * full, full)
                csl = pl.ds(s * cfull, cfull)
                pltpu.sync_copy(aggr_s.at[rsl], aggr_hbm.at[rsl])
                pltpu.sync_copy(cnt_s.at[csl], cnt_hbm.at[csl])
            @pl.when(s == TILES - 1)
            def _():
                last = N - (TILES - 1) * full
                rsl = pl.ds((TILES - 1) * full, last)
                csl = pl.ds((TILES - 1) * cfull, last * CNT_W)
                pltpu.sync_copy(aggr_s.at[rsl], aggr_hbm.at[rsl])
                pltpu.sync_copy(cnt_s.at[csl], cnt_hbm.at[csl])

        @pl.when(c == 0)
        def _():
            dump(aggr_a_hbm, cnt0_hbm)
        @pl.when(c == 1)
        def _():
            dump(aggr_b_hbm, cnt1_hbm)

    return k(ha, hb, src3, dst3, t3, d3)


def _tc_mlp(aggr_a, aggr_b, cnt0, cnt1, T0, T1, selfemb,
            W1T, b1r, W2T, b2r, WoutT, bor):
    def body(aa_ref, ab_ref, c0_ref, c1_ref, t0_ref, t1_ref,
             se_ref, w1_ref, b1_ref, w2_ref, b2_ref, wo_ref, bo_ref, o_ref,
             t0w_ref, t1w_ref, sb_ref, wc_ref, bc_ref):
        i = pl.program_id(0)

        @pl.when(i == 0)
        def _():
            w1 = w1_ref[...]
            t0w_ref[...] = jnp.dot(t0_ref[...], w1,
                                   preferred_element_type=jnp.float32)
            t1w_ref[...] = jnp.dot(t1_ref[...], w1,
                                   preferred_element_type=jnp.float32)
            sb_ref[...] = jnp.dot(se_ref[...], w1,
                                  preferred_element_type=jnp.float32) \
                + b1_ref[...]
            wo = wo_ref[...]
            wc_ref[...] = jnp.dot(w2_ref[...], wo,
                                  preferred_element_type=jnp.float32)
            bc_ref[...] = jnp.dot(b2_ref[...], wo,
                                  preferred_element_type=jnp.float32) \
                + bo_ref[...]

        g = jnp.dot(aa_ref[...], w1_ref[0:HH, :],
                    preferred_element_type=jnp.float32)
        g = g + jnp.dot(ab_ref[...], w1_ref[HH:H, :],
                        preferred_element_type=jnp.float32)
        g = g + jnp.dot(c0_ref[...], t0w_ref[...],
                        preferred_element_type=jnp.float32)
        g = g + jnp.dot(c1_ref[...], t1w_ref[...],
                        preferred_element_type=jnp.float32)
        g = jnp.maximum(g + sb_ref[...], 0.0)
        o_ref[...] = jnp.dot(g, wc_ref[...],
                             preferred_element_type=jnp.float32) + bc_ref[...]

    return pl.pallas_call(
        body,
        grid=(N // BLK,),
        in_specs=[
            pl.BlockSpec((BLK, HH), lambda i: (i, 0)),
            pl.BlockSpec((BLK, HH), lambda i: (i, 0)),
            pl.BlockSpec((BLK, CNT_W), lambda i: (i, 0)),
            pl.BlockSpec((BLK, CNT_W), lambda i: (i, 0)),
            pl.BlockSpec((CNT_W, H), lambda i: (0, 0)),
            pl.BlockSpec((CNT_W, H), lambda i: (0, 0)),
            pl.BlockSpec((1, H), lambda i: (0, 0)),
            pl.BlockSpec((H, 2 * H), lambda i: (0, 0)),
            pl.BlockSpec((1, 2 * H), lambda i: (0, 0)),
            pl.BlockSpec((2 * H, H), lambda i: (0, 0)),
            pl.BlockSpec((1, H), lambda i: (0, 0)),
            pl.BlockSpec((H, H), lambda i: (0, 0)),
            pl.BlockSpec((1, H), lambda i: (0, 0)),
        ],
        out_specs=pl.BlockSpec((BLK, H), lambda i: (i, 0)),
        out_shape=jax.ShapeDtypeStruct((N, H), jnp.float32),
        scratch_shapes=[
            pltpu.VMEM((CNT_W, 2 * H), jnp.float32),
            pltpu.VMEM((CNT_W, 2 * H), jnp.float32),
            pltpu.VMEM((1, 2 * H), jnp.float32),
            pltpu.VMEM((2 * H, H), jnp.float32),
            pltpu.VMEM((1, H), jnp.float32),
        ],
    )(aggr_a, aggr_b, cnt0, cnt1, T0, T1, selfemb,
      W1T, b1r, W2T, b2r, WoutT, bor)


def kernel(x, edge_index, edge_attr, batch, masked_node_indices, W_enc,
           prelu_a, dec_token, emb1, emb2, W1, b1, W2, b2, W_out, b_out):
    W_encT = W_enc.T
    a11 = jnp.reshape(prelu_a, (1, 1))
    masked2d = jnp.pad(masked_node_indices,
                       (0, MP - masked_node_indices.shape[0]),
                       constant_values=-1).reshape(MP // 128, 128)

    src = edge_index[0]
    dst = edge_index[1]
    pad_n = EP - E
    pad_iota = jnp.arange(pad_n, dtype=jnp.int32)
    src_p = jnp.concatenate([src, pad_iota % N])
    dst_p = jnp.concatenate([dst, N + (pad_iota % (NP - N))])
    t_p = jnp.concatenate([edge_attr[:, 0], jnp.zeros(pad_n, jnp.int32)])
    d_p = jnp.concatenate([edge_attr[:, 1], jnp.zeros(pad_n, jnp.int32)])
    src3 = src_p.reshape(TILES, CH, CW)
    dst3 = dst_p.reshape(TILES, CH, CW)
    t3 = t_p.reshape(TILES, CH, CW)
    d3 = d_p.reshape(TILES, CH, CW)

    T0 = jnp.zeros((CNT_W, H), jnp.float32).at[0:6].set(emb1)
    T1 = jnp.zeros((CNT_W, H), jnp.float32).at[0:3].set(emb2)
    selfemb = (emb1[4] + emb2[0]).reshape(1, H)

    W1T = W1.T
    b1r = b1.reshape(1, 2 * H)
    W2T = W2.T
    b2r = b2.reshape(1, H)
    WoutT = W_out.T
    bor = b_out.reshape(1, H)

    ha, hb = _tc_encode(x, W_encT, a11, dec_token, masked2d)

    aggr_a, aggr_b, cnt0_flat, cnt1_flat = _sc_aggregate(
        ha, hb, src3, dst3, t3, d3)
    cnt0 = cnt0_flat.reshape(N, CNT_W)
    cnt1 = cnt1_flat.reshape(N, CNT_W)

    return _tc_mlp(aggr_a, aggr_b, cnt0, cnt1, T0, T1, selfemb,
                   W1T, b1r, W2T, b2r, WoutT, bor)
